# Initial kernel scaffold; baseline (speedup 1.0000x reference)
import jax
import jax.numpy as jnp
from jax.experimental import pallas as pl


def kernel(hm, reg, wh, seg_feat, conv_weight):
    raise NotImplementedError("write your pallas kernel here")


# speedup vs baseline: 12.8251x; 12.8251x over previous
import jax
import jax.numpy as jnp
from jax.experimental import pallas as pl
from jax.experimental.pallas import tpu as pltpu

K_DET = 64


def _body(hm_ref, reg_ref, wh_ref, conv_ref, bb_ref, cv_ref, nms_ref):
    C, H, W = hm_ref.shape[1], hm_ref.shape[2], hm_ref.shape[3]
    CW = conv_ref.shape[1]
    x = hm_ref[0]
    s = jax.nn.sigmoid(x)
    ninf = jnp.float32(-jnp.inf)
    padh = jnp.full((C, 1, W), ninf, jnp.float32)
    v = jnp.maximum(s, jnp.concatenate([s[:, 1:, :], padh], axis=1))
    v = jnp.maximum(v, jnp.concatenate([padh, s[:, :-1, :]], axis=1))
    padw = jnp.full((C, H, 1), ninf, jnp.float32)
    hmax = jnp.maximum(v, jnp.concatenate([v[:, :, 1:], padw], axis=2))
    hmax = jnp.maximum(hmax, jnp.concatenate([padw, v[:, :, :-1]], axis=2))
    nmsed = jnp.where(s == hmax, s, jnp.float32(0.0))
    nms_ref[...] = nmsed
    l1_0 = jnp.max(nmsed, axis=2)

    flat_ci = (jax.lax.broadcasted_iota(jnp.int32, (C, H), 0) * H
               + jax.lax.broadcasted_iota(jnp.int32, (C, H), 1))
    jiota = jax.lax.broadcasted_iota(jnp.int32, (1, W), 1)
    kiota = jax.lax.broadcasted_iota(jnp.int32, (1, K_DET), 1)
    krows = jax.lax.broadcasted_iota(jnp.int32, (K_DET, 1), 0)
    big = jnp.int32(1 << 30)

    def pick_lane(rowvec, j):
        return jnp.sum(jnp.where(jiota == j, rowvec, jnp.float32(0.0)))

    def step(k, carry):
        l1, bx1, by1, bx2, by2, bv, bc, pcol = carry
        m = jnp.max(l1)
        f = jnp.min(jnp.where(l1 == m, flat_ci, big))
        c = jax.lax.div(f, jnp.int32(H))
        i = jax.lax.rem(f, jnp.int32(H))
        row = nms_ref[c, pl.ds(i, 1), :]
        j = jnp.min(jnp.where(row == m, jiota, big))
        p = i * W + j
        newrow = jnp.where(jiota == j, jnp.float32(-1.0), row)
        nms_ref[c, pl.ds(i, 1), :] = newrow
        l1 = jnp.where(flat_ci == f, jnp.max(newrow), l1)
        r0 = pick_lane(reg_ref[0, 0, pl.ds(i, 1), :], j)
        r1 = pick_lane(reg_ref[0, 1, pl.ds(i, 1), :], j)
        w0 = pick_lane(wh_ref[0, 0, pl.ds(i, 1), :], j)
        w1 = pick_lane(wh_ref[0, 1, pl.ds(i, 1), :], j)
        xc = j.astype(jnp.float32) + r0
        yc = i.astype(jnp.float32) + r1
        sel = kiota == k
        bx1 = jnp.where(sel, xc - w0 * 0.5, bx1)
        by1 = jnp.where(sel, yc - w1 * 0.5, by1)
        bx2 = jnp.where(sel, xc + w0 * 0.5, bx2)
        by2 = jnp.where(sel, yc + w1 * 0.5, by2)
        bv = jnp.where(sel, m, bv)
        bc = jnp.where(sel, c.astype(jnp.float32), bc)
        pcol = jnp.where(krows == k, p, pcol)
        return (l1, bx1, by1, bx2, by2, bv, bc, pcol)

    z = jnp.zeros((1, K_DET), jnp.float32)
    carry0 = (l1_0, z, z, z, z, z, z, jnp.zeros((K_DET, 1), jnp.int32))
    out = jax.lax.fori_loop(0, K_DET, step, carry0)
    _, bx1, by1, bx2, by2, bv, bc, pcol = out

    bb_ref[0] = jnp.concatenate(
        [bx1, by1, bx2, by2, bv, bc, jnp.zeros((2, K_DET), jnp.float32)],
        axis=0)

    onehot = (jax.lax.broadcasted_iota(jnp.int32, (K_DET, H * W), 1)
              == pcol).astype(jnp.float32)
    convr = conv_ref[0].reshape(CW, H * W)
    cv = jax.lax.dot_general(onehot, convr, (((1,), (1,)), ((), ())),
                             precision=jax.lax.Precision.HIGHEST)
    cv_ref[0] = cv


def kernel(hm, reg, wh, seg_feat, conv_weight):
    B, C, H, W = hm.shape
    CW = conv_weight.shape[1]
    bb, cv = pl.pallas_call(
        _body,
        grid=(B,),
        in_specs=[
            pl.BlockSpec((1, C, H, W), lambda b: (b, 0, 0, 0)),
            pl.BlockSpec((1, 2, H, W), lambda b: (b, 0, 0, 0)),
            pl.BlockSpec((1, 2, H, W), lambda b: (b, 0, 0, 0)),
            pl.BlockSpec((1, CW, H, W), lambda b: (b, 0, 0, 0)),
        ],
        out_specs=[
            pl.BlockSpec((1, 8, K_DET), lambda b: (b, 0, 0)),
            pl.BlockSpec((1, K_DET, CW), lambda b: (b, 0, 0)),
        ],
        out_shape=[
            jax.ShapeDtypeStruct((B, 8, K_DET), jnp.float32),
            jax.ShapeDtypeStruct((B, K_DET, CW), jnp.float32),
        ],
        scratch_shapes=[pltpu.VMEM((C, H, W), jnp.float32)],
    )(hm, reg, wh, conv_weight)
    bboxes = jnp.transpose(bb[:, 0:6, :], (0, 2, 1))
    return (bboxes, seg_feat, cv)


# speedup vs baseline: 13.0211x; 1.0153x over previous
import jax
import jax.numpy as jnp
from jax.experimental import pallas as pl
from jax.experimental.pallas import tpu as pltpu

K_DET = 64


def _body(hm_ref, reg_ref, wh_ref, conv_ref, bb_ref, cv_ref,
          nms_ref, g_ref, v_ref):
    C, H, W = hm_ref.shape[1], hm_ref.shape[2], hm_ref.shape[3]
    CW = conv_ref.shape[1]
    x = hm_ref[0]
    s = jax.nn.sigmoid(x)
    ninf = jnp.float32(-jnp.inf)
    padh = jnp.full((C, 1, W), ninf, jnp.float32)
    v = jnp.maximum(s, jnp.concatenate([s[:, 1:, :], padh], axis=1))
    v = jnp.maximum(v, jnp.concatenate([padh, s[:, :-1, :]], axis=1))
    padw = jnp.full((C, H, 1), ninf, jnp.float32)
    hmax = jnp.maximum(v, jnp.concatenate([v[:, :, 1:], padw], axis=2))
    hmax = jnp.maximum(hmax, jnp.concatenate([padw, v[:, :, :-1]], axis=2))
    nmsed = jnp.where(s == hmax, s, jnp.float32(0.0))
    nms_ref[...] = nmsed
    l1_0 = jnp.max(nmsed, axis=2)

    flat_ci = (jax.lax.broadcasted_iota(jnp.int32, (C, H), 0) * H
               + jax.lax.broadcasted_iota(jnp.int32, (C, H), 1))
    jiota = jax.lax.broadcasted_iota(jnp.int32, (1, W), 1)
    kiota = jax.lax.broadcasted_iota(jnp.int32, (1, K_DET), 1)
    big = jnp.int32(1 << 30)

    def step(k, l1):
        m = jnp.max(l1)
        f = jnp.min(jnp.where(l1 == m, flat_ci, big))
        c = jax.lax.shift_right_logical(f, 7)
        i = jax.lax.bitwise_and(f, jnp.int32(H - 1))
        row = nms_ref[c, pl.ds(i, 1), :]
        j = jnp.min(jnp.where(row == m, jiota, big))
        newrow = jnp.where(jiota == j, jnp.float32(-1.0), row)
        nms_ref[c, pl.ds(i, 1), :] = newrow
        l1 = jnp.where(flat_ci == f, jnp.max(newrow), l1)
        g_ref[k] = (c * H + i) * W + j
        v_ref[k] = m
        return l1

    jax.lax.fori_loop(0, K_DET, step, l1_0)

    z = jnp.zeros((1, K_DET), jnp.float32)
    bx1 = by1 = bx2 = by2 = bv = bc = z
    half = jnp.float32(0.5)
    for k in range(K_DET):
        g = g_ref[k]
        m = v_ref[k]
        c = jax.lax.shift_right_logical(g, 14)
        i = jax.lax.bitwise_and(jax.lax.shift_right_logical(g, 7),
                                jnp.int32(H - 1))
        j = jax.lax.bitwise_and(g, jnp.int32(W - 1))
        jm = jiota == j
        r0 = jnp.sum(jnp.where(jm, reg_ref[0, 0, pl.ds(i, 1), :], 0.0))
        r1 = jnp.sum(jnp.where(jm, reg_ref[0, 1, pl.ds(i, 1), :], 0.0))
        w0 = jnp.sum(jnp.where(jm, wh_ref[0, 0, pl.ds(i, 1), :], 0.0))
        w1 = jnp.sum(jnp.where(jm, wh_ref[0, 1, pl.ds(i, 1), :], 0.0))
        col = conv_ref[0, :, pl.ds(i, 1), :]
        ck = jnp.sum(jnp.where(jm[None], col, 0.0), axis=2)
        cv_ref[0, :, k:k + 1] = ck
        xc = j.astype(jnp.float32) + r0
        yc = i.astype(jnp.float32) + r1
        sel = kiota == k
        bx1 = jnp.where(sel, xc - w0 * half, bx1)
        by1 = jnp.where(sel, yc - w1 * half, by1)
        bx2 = jnp.where(sel, xc + w0 * half, bx2)
        by2 = jnp.where(sel, yc + w1 * half, by2)
        bv = jnp.where(sel, m, bv)
        bc = jnp.where(sel, c.astype(jnp.float32), bc)

    bb_ref[0] = jnp.concatenate(
        [bx1, by1, bx2, by2, bv, bc, jnp.zeros((2, K_DET), jnp.float32)],
        axis=0)


def kernel(hm, reg, wh, seg_feat, conv_weight):
    B, C, H, W = hm.shape
    CW = conv_weight.shape[1]
    bb, cv = pl.pallas_call(
        _body,
        grid=(B,),
        in_specs=[
            pl.BlockSpec((1, C, H, W), lambda b: (b, 0, 0, 0)),
            pl.BlockSpec((1, 2, H, W), lambda b: (b, 0, 0, 0)),
            pl.BlockSpec((1, 2, H, W), lambda b: (b, 0, 0, 0)),
            pl.BlockSpec((1, CW, H, W), lambda b: (b, 0, 0, 0)),
        ],
        out_specs=[
            pl.BlockSpec((1, 8, K_DET), lambda b: (b, 0, 0)),
            pl.BlockSpec((1, CW, K_DET), lambda b: (b, 0, 0)),
        ],
        out_shape=[
            jax.ShapeDtypeStruct((B, 8, K_DET), jnp.float32),
            jax.ShapeDtypeStruct((B, CW, K_DET), jnp.float32),
        ],
        scratch_shapes=[
            pltpu.VMEM((C, H, W), jnp.float32),
            pltpu.SMEM((K_DET,), jnp.int32),
            pltpu.SMEM((K_DET,), jnp.float32),
        ],
    )(hm, reg, wh, conv_weight)
    bboxes = jnp.transpose(bb[:, 0:6, :], (0, 2, 1))
    conv_weights = jnp.transpose(cv, (0, 2, 1))
    return (bboxes, seg_feat, conv_weights)
